# baseline (speedup 1.0000x reference)
import functools

import jax
import jax.numpy as jnp
from jax import lax
from jax.experimental import pallas as pl
from jax.experimental.pallas import tpu as pltpu
from jax.experimental.pallas import tpu_sc as plsc

_N = 320000
_D = 128
_K = 100
_KP = 128
_BLK = 16000
_NB = _N // _BLK

_NW = 32
_RW = _N // _NW
_CH = 400
_NCH = _RW // _CH


@functools.partial(
    pl.kernel,
    mesh=plsc.VectorSubcoreMesh(core_axis_name="c", subcore_axis_name="s"),
    out_type=jax.ShapeDtypeStruct((_NW, _KP, _D), jnp.float32),
    scratch_types=[
        pltpu.VMEM((_CH, _D), jnp.float32),
        pltpu.VMEM((_CH,), jnp.int32),
        pltpu.VMEM((_KP, _D), jnp.float32),
    ],
)
def _sc_segment_sums(x_hbm, lab_hbm, out_hbm, xbuf, labbuf, sums_v):
    wid = lax.axis_index("s") * 2 + lax.axis_index("c")
    base = wid * _RW

    def _zero_row(r, carry):
        for j in range(_D // 16):
            sums_v[r, pl.ds(j * 16, 16)] = jnp.zeros((16,), jnp.float32)
        return carry

    lax.fori_loop(0, _KP, _zero_row, 0)

    def _chunk(ch, carry):
        off = base + ch * _CH
        pltpu.sync_copy(x_hbm.at[pl.ds(off, _CH)], xbuf)
        pltpu.sync_copy(lab_hbm.at[pl.ds(off, _CH)], labbuf)

        def _group(g, c2):
            lab16 = labbuf[pl.ds(g * 16, 16)]
            for q in range(16):
                lbl = lab16[q]
                r = g * 16 + q
                for j in range(_D // 16):
                    sl = pl.ds(j * 16, 16)
                    plsc.addupdate(sums_v.at[lbl, sl], xbuf[r, sl])
            return c2

        lax.fori_loop(0, _CH // 16, _group, 0)
        return carry

    lax.fori_loop(0, _NCH, _chunk, 0)
    pltpu.sync_copy(sums_v, out_hbm.at[wid])


def _loss_kernel(x_ref, lab_ref, part_ref, out_ref, sums_ref, counts_ref,
                 centers_ref, acc_ref):
    p = pl.program_id(0)
    i = pl.program_id(1)

    lab = lab_ref[0, 0, :]
    ids = jax.lax.broadcasted_iota(jnp.int16, (_BLK, _KP), 1)
    oh = jnp.where(lab.astype(jnp.int16)[:, None] == ids,
                   jnp.bfloat16(1), jnp.bfloat16(0))

    @pl.when(jnp.logical_and(p == 0, i == 0))
    def _init():
        counts_ref[...] = jnp.zeros_like(counts_ref)

    @pl.when(p == 0)
    def _counts():
        ones_row = jnp.ones((1, _BLK), jnp.bfloat16)
        counts_ref[...] += jax.lax.dot_general(
            ones_row, oh, (((1,), (0,)), ((), ())),
            preferred_element_type=jnp.float32)

    @pl.when(jnp.logical_and(p == 1, i == 0))
    def _centers_and_inter():
        sums_ref[...] = jnp.sum(part_ref[...], axis=0)
        cnt = counts_ref[0, :]
        inv = jnp.where(cnt > 0.0, 1.0 / cnt, 0.0)
        centers = sums_ref[...] * inv[:, None]
        centers_ref[...] = centers
        g = jax.lax.dot_general(
            centers, centers, (((1,), (1,)), ((), ())),
            preferred_element_type=jnp.float32,
            precision=jax.lax.Precision.HIGHEST)
        n2 = jnp.sum(centers * centers, axis=1)
        d2 = n2[:, None] + n2[None, :] - 2.0 * g
        r = jax.lax.broadcasted_iota(jnp.int32, (_KP, _KP), 0)
        c = jax.lax.broadcasted_iota(jnp.int32, (_KP, _KP), 1)
        valid = jnp.logical_and(r < c, c < _K)
        d = jnp.sqrt(jnp.maximum(d2, 0.0))
        acc_ref[0] = jnp.sum(jnp.where(valid, d, 0.0))
        acc_ref[1] = 0.0

    @pl.when(p == 1)
    def _intra():
        x_bf = x_ref[...].astype(jnp.bfloat16)
        centers_bf = centers_ref[...].astype(jnp.bfloat16)
        cg = jax.lax.dot_general(
            oh, centers_bf, (((1,), (0,)), ((), ())),
            preferred_element_type=jnp.float32).astype(jnp.bfloat16)
        diff = x_bf - cg
        sq = diff * diff
        ones_row = jnp.ones((1, _D), jnp.bfloat16)
        rs = jax.lax.dot_general(
            ones_row, sq, (((1,), (1,)), ((), ())),
            preferred_element_type=jnp.float32)[0]
        acc_ref[1] += jnp.sum(jnp.sqrt(rs))

        @pl.when(i == _NB - 1)
        def _fin():
            inter = acc_ref[0]
            intra = acc_ref[1]
            out_ref[0, 0] = jnp.where(inter > 0.0, intra / inter, intra)


def kernel(x, labels):
    labels_i32 = labels.astype(jnp.int32)
    partials = _sc_segment_sums(x, labels_i32)
    labels3 = labels_i32.reshape(_NB, 1, _BLK)
    out = pl.pallas_call(
        _loss_kernel,
        grid=(2, _NB),
        in_specs=[
            pl.BlockSpec((_BLK, _D), lambda p, i: (i * p, 0)),
            pl.BlockSpec((1, 1, _BLK), lambda p, i: (i, 0, 0)),
            pl.BlockSpec((_NW, _KP, _D), lambda p, i: (0, 0, 0)),
        ],
        out_shape=jax.ShapeDtypeStruct((1, 1), jnp.float32),
        out_specs=pl.BlockSpec(memory_space=pltpu.SMEM),
        scratch_shapes=[
            pltpu.VMEM((_KP, _D), jnp.float32),
            pltpu.VMEM((1, _KP), jnp.float32),
            pltpu.VMEM((_KP, _D), jnp.float32),
            pltpu.SMEM((2,), jnp.float32),
        ],
    )(x, labels3, partials)
    return out[0, 0]


# speedup vs baseline: 1.4760x; 1.4760x over previous
import functools

import jax
import jax.numpy as jnp
from jax import lax
from jax.experimental import pallas as pl
from jax.experimental.pallas import tpu as pltpu
from jax.experimental.pallas import tpu_sc as plsc

_N = 320000
_D = 128
_K = 100
_KP = 128
_BLK = 16000
_NB = _N // _BLK

_NW = 32
_RW = _N // _NW
_CH = 400
_NCH = _RW // _CH


@functools.partial(
    pl.kernel,
    mesh=plsc.VectorSubcoreMesh(core_axis_name="c", subcore_axis_name="s"),
    out_type=jax.ShapeDtypeStruct((_NW, _KP, _D), jnp.float32),
    scratch_types=[
        pltpu.VMEM((_CH, _D), jnp.float32),
        pltpu.VMEM((_CH,), jnp.int32),
        pltpu.VMEM((_KP, _D), jnp.float32),
    ],
)
def _sc_segment_sums(x_hbm, lab_hbm, out_hbm, xbuf, labbuf, sums_v):
    wid = lax.axis_index("s") * 2 + lax.axis_index("c")
    base = wid * _RW

    def _zero_row(r, carry):
        for j in range(_D // 16):
            sums_v[r, pl.ds(j * 16, 16)] = jnp.zeros((16,), jnp.float32)
        return carry

    lax.fori_loop(0, _KP, _zero_row, 0)

    def _chunk(ch, carry):
        off = base + ch * _CH
        pltpu.sync_copy(x_hbm.at[pl.ds(off, _CH)], xbuf)
        pltpu.sync_copy(lab_hbm.at[pl.ds(off, _CH)], labbuf)

        def _group(g, c2):
            lab16 = labbuf[pl.ds(g * 16, 16)]
            l0 = lab16[0]

            def _uniform(_):
                for j in range(_D // 16):
                    sl = pl.ds(j * 16, 16)
                    acc = xbuf[g * 16, sl]
                    for q in range(1, 16):
                        acc = acc + xbuf[g * 16 + q, sl]
                    plsc.addupdate(sums_v.at[l0, sl], acc)
                return 0

            def _mixed(_):
                for q in range(16):
                    lbl = lab16[q]
                    r = g * 16 + q
                    for j in range(_D // 16):
                        sl = pl.ds(j * 16, 16)
                        plsc.addupdate(sums_v.at[lbl, sl], xbuf[r, sl])
                return 0

            lax.cond(l0 == lab16[15], _uniform, _mixed, 0)
            return c2

        lax.fori_loop(0, _CH // 16, _group, 0)
        return carry

    lax.fori_loop(0, _NCH, _chunk, 0)
    pltpu.sync_copy(sums_v, out_hbm.at[wid])


def _loss_kernel(x_ref, lab_ref, part_ref, out_ref, sums_ref, counts_ref,
                 centers_ref, acc_ref):
    p = pl.program_id(0)
    i = pl.program_id(1)

    lab = lab_ref[0, 0, :]
    ids = jax.lax.broadcasted_iota(jnp.int16, (_BLK, _KP), 1)
    oh = jnp.where(lab.astype(jnp.int16)[:, None] == ids,
                   jnp.bfloat16(1), jnp.bfloat16(0))

    @pl.when(jnp.logical_and(p == 0, i == 0))
    def _init():
        counts_ref[...] = jnp.zeros_like(counts_ref)

    @pl.when(p == 0)
    def _counts():
        ones_row = jnp.ones((1, _BLK), jnp.bfloat16)
        counts_ref[...] += jax.lax.dot_general(
            ones_row, oh, (((1,), (0,)), ((), ())),
            preferred_element_type=jnp.float32)

    @pl.when(jnp.logical_and(p == 1, i == 0))
    def _centers_and_inter():
        sums_ref[...] = jnp.sum(part_ref[...], axis=0)
        cnt = counts_ref[0, :]
        inv = jnp.where(cnt > 0.0, 1.0 / cnt, 0.0)
        centers = sums_ref[...] * inv[:, None]
        centers_ref[...] = centers
        g = jax.lax.dot_general(
            centers, centers, (((1,), (1,)), ((), ())),
            preferred_element_type=jnp.float32,
            precision=jax.lax.Precision.HIGHEST)
        n2 = jnp.sum(centers * centers, axis=1)
        d2 = n2[:, None] + n2[None, :] - 2.0 * g
        r = jax.lax.broadcasted_iota(jnp.int32, (_KP, _KP), 0)
        c = jax.lax.broadcasted_iota(jnp.int32, (_KP, _KP), 1)
        valid = jnp.logical_and(r < c, c < _K)
        d = jnp.sqrt(jnp.maximum(d2, 0.0))
        acc_ref[0] = jnp.sum(jnp.where(valid, d, 0.0))
        acc_ref[1] = 0.0

    @pl.when(p == 1)
    def _intra():
        x_bf = x_ref[...].astype(jnp.bfloat16)
        centers_bf = centers_ref[...].astype(jnp.bfloat16)
        cg = jax.lax.dot_general(
            oh, centers_bf, (((1,), (0,)), ((), ())),
            preferred_element_type=jnp.float32).astype(jnp.bfloat16)
        diff = x_bf - cg
        sq = diff * diff
        ones_row = jnp.ones((1, _D), jnp.bfloat16)
        rs = jax.lax.dot_general(
            ones_row, sq, (((1,), (1,)), ((), ())),
            preferred_element_type=jnp.float32)[0]
        acc_ref[1] += jnp.sum(jnp.sqrt(rs))

        @pl.when(i == _NB - 1)
        def _fin():
            inter = acc_ref[0]
            intra = acc_ref[1]
            out_ref[0, 0] = jnp.where(inter > 0.0, intra / inter, intra)


def kernel(x, labels):
    labels_i32 = labels.astype(jnp.int32)
    partials = _sc_segment_sums(x, labels_i32)
    labels3 = labels_i32.reshape(_NB, 1, _BLK)
    out = pl.pallas_call(
        _loss_kernel,
        grid=(2, _NB),
        in_specs=[
            pl.BlockSpec((_BLK, _D), lambda p, i: (i * p, 0)),
            pl.BlockSpec((1, 1, _BLK), lambda p, i: (i, 0, 0)),
            pl.BlockSpec((_NW, _KP, _D), lambda p, i: (0, 0, 0)),
        ],
        out_shape=jax.ShapeDtypeStruct((1, 1), jnp.float32),
        out_specs=pl.BlockSpec(memory_space=pltpu.SMEM),
        scratch_shapes=[
            pltpu.VMEM((_KP, _D), jnp.float32),
            pltpu.VMEM((1, _KP), jnp.float32),
            pltpu.VMEM((_KP, _D), jnp.float32),
            pltpu.SMEM((2,), jnp.float32),
        ],
    )(x, labels3, partials)
    return out[0, 0]


# speedup vs baseline: 3.2304x; 2.1886x over previous
import jax
import jax.numpy as jnp
from jax.experimental import pallas as pl
from jax.experimental.pallas import tpu as pltpu

_N = 320000
_D = 128
_K = 100
_KP = 128
_BLK = 20000
_NB = _N // _BLK


def _loss_kernel(x_ref, lab_ref, out_ref, sums_ref, counts_ref, centers_ref,
                 acc_ref):
    p = pl.program_id(0)
    i = pl.program_id(1)

    lab = lab_ref[0, 0, :]
    ids = jax.lax.broadcasted_iota(jnp.int16, (_BLK, _KP), 1)
    oh = jnp.where(lab.astype(jnp.int16)[:, None] == ids,
                   jnp.bfloat16(1), jnp.bfloat16(0))
    x_bf = x_ref[...].astype(jnp.bfloat16)

    @pl.when(jnp.logical_and(p == 0, i == 0))
    def _init():
        sums_ref[...] = jnp.zeros_like(sums_ref)
        counts_ref[...] = jnp.zeros_like(counts_ref)

    @pl.when(p == 0)
    def _accum():
        sums_ref[...] += jax.lax.dot_general(
            oh, x_bf, (((0,), (0,)), ((), ())),
            preferred_element_type=jnp.float32)
        ones_row = jnp.ones((1, _BLK), jnp.bfloat16)
        counts_ref[...] += jax.lax.dot_general(
            ones_row, oh, (((1,), (0,)), ((), ())),
            preferred_element_type=jnp.float32)

    @pl.when(jnp.logical_and(p == 1, i == 0))
    def _centers_and_inter():
        cnt = counts_ref[0, :]
        inv = jnp.where(cnt > 0.0, 1.0 / cnt, 0.0)
        centers = sums_ref[...] * inv[:, None]
        centers_ref[...] = centers
        g = jax.lax.dot_general(
            centers, centers, (((1,), (1,)), ((), ())),
            preferred_element_type=jnp.float32,
            precision=jax.lax.Precision.HIGHEST)
        n2 = jnp.sum(centers * centers, axis=1)
        d2 = n2[:, None] + n2[None, :] - 2.0 * g
        r = jax.lax.broadcasted_iota(jnp.int32, (_KP, _KP), 0)
        c = jax.lax.broadcasted_iota(jnp.int32, (_KP, _KP), 1)
        valid = jnp.logical_and(r < c, c < _K)
        d = jnp.sqrt(jnp.maximum(d2, 0.0))
        acc_ref[0] = jnp.sum(jnp.where(valid, d, 0.0))
        acc_ref[1] = 0.0

    @pl.when(p == 1)
    def _intra():
        centers_bf = centers_ref[...].astype(jnp.bfloat16)
        cg = jax.lax.dot_general(
            oh, centers_bf, (((1,), (0,)), ((), ())),
            preferred_element_type=jnp.float32).astype(jnp.bfloat16)
        diff = x_bf - cg
        sq = diff * diff
        ones_row = jnp.ones((1, _D), jnp.bfloat16)
        rs = jax.lax.dot_general(
            ones_row, sq, (((1,), (1,)), ((), ())),
            preferred_element_type=jnp.float32)[0]
        acc_ref[1] += jnp.sum(jnp.sqrt(rs))

        @pl.when(i == _NB - 1)
        def _fin():
            inter = acc_ref[0]
            intra = acc_ref[1]
            out_ref[0, 0] = jnp.where(inter > 0.0, intra / inter, intra)


def kernel(x, labels):
    labels3 = labels.astype(jnp.int32).reshape(_NB, 1, _BLK)
    out = pl.pallas_call(
        _loss_kernel,
        grid=(2, _NB),
        in_specs=[
            pl.BlockSpec((_BLK, _D), lambda p, i: (i, 0)),
            pl.BlockSpec((1, 1, _BLK), lambda p, i: (i, 0, 0)),
        ],
        out_shape=jax.ShapeDtypeStruct((1, 1), jnp.float32),
        out_specs=pl.BlockSpec(memory_space=pltpu.SMEM),
        scratch_shapes=[
            pltpu.VMEM((_KP, _D), jnp.float32),
            pltpu.VMEM((1, _KP), jnp.float32),
            pltpu.VMEM((_KP, _D), jnp.float32),
            pltpu.SMEM((2,), jnp.float32),
        ],
    )(x, labels3)
    return out[0, 0]
